# baseline (speedup 1.0000x reference)
import jax
import jax.numpy as jnp
from jax import lax
from jax.experimental import pallas as pl
from jax.experimental.pallas import tpu as pltpu

R = 1024


def _tc_body(x_ref, idx_ref, emb_ref, o_ref):
    idxb = jnp.maximum(idx_ref[0, 0, :], 0)
    oh = (idxb[:, None] == lax.broadcasted_iota(jnp.int32, (1, 12), 1)
          ).astype(jnp.float32)
    e = jnp.dot(oh, emb_ref[...], preferred_element_type=jnp.float32)
    o_ref[...] = x_ref[...] + e


def kernel(x, month_idx, emb):
    b, l, d = x.shape
    n_rows = b * l
    grid = n_rows // R
    assert grid * R == n_rows

    x2 = x.reshape(n_rows, d)
    idx = month_idx.reshape(grid, 1, R).astype(jnp.int32)

    out = pl.pallas_call(
        _tc_body,
        grid=(grid,),
        in_specs=[
            pl.BlockSpec((R, d), lambda i: (i, 0)),
            pl.BlockSpec((1, 1, R), lambda i: (i, 0, 0)),
            pl.BlockSpec(emb.shape, lambda i: (0, 0)),
        ],
        out_specs=pl.BlockSpec((R, d), lambda i: (i, 0)),
        out_shape=jax.ShapeDtypeStruct((n_rows, d), jnp.float32),
    )(x2, idx, emb)
    return out.reshape(b, l, d)


# speedup vs baseline: 1.5778x; 1.5778x over previous
import jax
import jax.numpy as jnp
from jax import lax
from jax.experimental import pallas as pl
from jax.experimental.pallas import tpu as pltpu
from jax.experimental.pallas import tpu_sc as plsc

NC = 2
NS = 16
NW = NC * NS
LANES = 16
CHUNK = 80
NBUF = 4


def _sc_body(rows_per_w, n_chunks, d, x_hbm, idx_hbm, emb_hbm, out_hbm,
             emb_v, idx_all, bufs, sp, in_sems, sp_sems, out_sems):
    wid = lax.axis_index("s") * NC + lax.axis_index("c")
    sid = lax.axis_index("s")
    base = wid * rows_per_w
    lookahead = NBUF - 1

    pltpu.sync_copy(emb_hbm, emb_v)
    pltpu.sync_copy(idx_hbm.at[pl.ds(base, rows_per_w)], idx_all)

    def start_in(k, b):
        row0 = base + k * CHUNK
        pltpu.async_copy(x_hbm.at[pl.ds(row0, CHUNK)], bufs[b], in_sems[b])

    def wait_in(b):
        pltpu.make_async_copy(x_hbm.at[pl.ds(base, CHUNK)], bufs[b],
                              in_sems[b]).wait()

    def start_sp(b):
        pltpu.async_copy(bufs[b], sp.at[sid, b], sp_sems[b])

    def wait_sp(b):
        pltpu.make_async_copy(bufs[b], sp.at[sid, b], sp_sems[b]).wait()

    def start_out(k, b):
        row0 = base + k * CHUNK
        pltpu.async_copy(sp.at[sid, b], out_hbm.at[pl.ds(row0, CHUNK)],
                         out_sems[b])

    def wait_out(b):
        pltpu.make_async_copy(sp.at[sid, b], out_hbm.at[pl.ds(base, CHUNK)],
                              out_sems[b]).wait()

    def compute(k, b):
        buf = bufs[b]
        idx0 = k * CHUNK

        @pl.loop(0, CHUNK // LANES)
        def _grp(g):
            idxv = jnp.maximum(idx_all[pl.ds(idx0 + g * LANES, LANES)], 0)
            for i in range(LANES):
                s = idxv[i]
                r = g * LANES + i
                sls = [pl.ds(j * LANES, LANES) for j in range(d // LANES)]
                es = [emb_v[s, sl] for sl in sls]
                for j, sl in enumerate(sls):
                    plsc.addupdate(buf.at[r, sl], es[j])

    def iter_body(k, b, static_tail):
        wait_in(b)
        compute(k, b)

        @pl.when(k >= NBUF)
        def _():
            wait_out(b)

        start_sp(b)
        wait_sp(b)
        start_out(k, b)
        if static_tail:
            return
        nxt = k + lookahead
        bn = (b + lookahead) % NBUF

        @pl.when(nxt < n_chunks)
        def _():
            start_in(nxt, bn)

    n_main = (n_chunks // NBUF) * NBUF

    for c in range(min(lookahead, n_chunks)):
        start_in(c, c % NBUF)

    @pl.loop(0, n_main, step=NBUF)
    def _kk(kk):
        for b in range(NBUF):
            iter_body(kk + b, b, False)

    for k in range(n_main, n_chunks):
        iter_body(k, k % NBUF, True)

    for i in range(min(NBUF, n_chunks)):
        wait_out((n_chunks - 1 - i) % NBUF)


def kernel(x, month_idx, emb):
    b, l, d = x.shape
    n_rows = b * l
    rows_per_w = n_rows // NW
    n_chunks = rows_per_w // CHUNK
    assert rows_per_w * NW == n_rows and n_chunks * CHUNK == rows_per_w

    x2 = x.reshape(n_rows, d)
    idx = month_idx.reshape(n_rows).astype(jnp.int32)

    mesh = plsc.VectorSubcoreMesh(core_axis_name="c", subcore_axis_name="s")
    body = lambda *refs: _sc_body(rows_per_w, n_chunks, d, *refs)
    out = pl.kernel(
        body,
        out_type=jax.ShapeDtypeStruct((n_rows, d), jnp.float32),
        mesh=mesh,
        scratch_types=[
            pltpu.VMEM((emb.shape[0], d), jnp.float32),
            pltpu.VMEM((rows_per_w,), jnp.int32),
            [pltpu.VMEM((CHUNK, d), jnp.float32) for _ in range(NBUF)],
            pltpu.VMEM_SHARED((NS, NBUF, CHUNK, d), jnp.float32),
            [pltpu.SemaphoreType.DMA for _ in range(NBUF)],
            [pltpu.SemaphoreType.DMA for _ in range(NBUF)],
            [pltpu.SemaphoreType.DMA for _ in range(NBUF)],
        ],
    )(x2, idx, emb)
    return out.reshape(b, l, d)


# speedup vs baseline: 2.0287x; 1.2858x over previous
import jax
import jax.numpy as jnp
from jax import lax
from jax.experimental import pallas as pl
from jax.experimental.pallas import tpu as pltpu
from jax.experimental.pallas import tpu_sc as plsc

NC = 2
NS = 16
NW = NC * NS
LANES = 16
CHUNK = 80
NBUF = 5


def _sc_body(rows_per_w, n_chunks, d, x_hbm, idx_hbm, emb_hbm, out_hbm,
             emb_v, idx_all, bufs, in_sems, out_sems):
    wid = lax.axis_index("s") * NC + lax.axis_index("c")
    base = wid * rows_per_w
    lookahead = NBUF - 1

    pltpu.sync_copy(emb_hbm, emb_v)
    pltpu.sync_copy(idx_hbm.at[pl.ds(base, rows_per_w)], idx_all)

    def start_in(k, b):
        row0 = base + k * CHUNK
        pltpu.async_copy(x_hbm.at[pl.ds(row0, CHUNK)], bufs[b], in_sems[b])

    def wait_in(b):
        pltpu.make_async_copy(x_hbm.at[pl.ds(base, CHUNK)], bufs[b],
                              in_sems[b]).wait()

    def start_out(k, b):
        row0 = base + k * CHUNK
        pltpu.async_copy(bufs[b], out_hbm.at[pl.ds(row0, CHUNK)],
                         out_sems[b])

    def wait_out(b):
        pltpu.make_async_copy(bufs[b], out_hbm.at[pl.ds(base, CHUNK)],
                              out_sems[b]).wait()

    def compute(k, b):
        buf = bufs[b]
        idx0 = k * CHUNK

        @pl.loop(0, CHUNK // LANES)
        def _grp(g):
            idxv = jnp.maximum(idx_all[pl.ds(idx0 + g * LANES, LANES)], 0)
            for i in range(LANES):
                s = idxv[i]
                r = g * LANES + i
                sls = [pl.ds(j * LANES, LANES) for j in range(d // LANES)]
                es = [emb_v[s, sl] for sl in sls]
                for j, sl in enumerate(sls):
                    plsc.addupdate(buf.at[r, sl], es[j])

    def iter_body(k, b, static_tail):
        wait_in(b)
        compute(k, b)
        start_out(k, b)
        if static_tail:
            return
        nxt = k + lookahead
        bn = (b + lookahead) % NBUF

        @pl.when(nxt < n_chunks)
        def _():
            @pl.when(k >= 1)
            def _():
                wait_out(bn)

            start_in(nxt, bn)

    n_main = (n_chunks // NBUF) * NBUF

    for c in range(min(lookahead, n_chunks)):
        start_in(c, c % NBUF)

    @pl.loop(0, n_main, step=NBUF)
    def _kk(kk):
        for b in range(NBUF):
            iter_body(kk + b, b, False)

    for k in range(n_main, n_chunks):
        iter_body(k, k % NBUF, True)

    for i in range(min(NBUF, n_chunks)):
        wait_out((n_chunks - 1 - i) % NBUF)


def kernel(x, month_idx, emb):
    b, l, d = x.shape
    n_rows = b * l
    rows_per_w = n_rows // NW
    n_chunks = rows_per_w // CHUNK
    assert rows_per_w * NW == n_rows and n_chunks * CHUNK == rows_per_w

    x2 = x.reshape(n_rows, d)
    idx = month_idx.reshape(n_rows).astype(jnp.int32)

    mesh = plsc.VectorSubcoreMesh(core_axis_name="c", subcore_axis_name="s")
    body = lambda *refs: _sc_body(rows_per_w, n_chunks, d, *refs)
    out = pl.kernel(
        body,
        out_type=jax.ShapeDtypeStruct((n_rows, d), jnp.float32),
        mesh=mesh,
        scratch_types=[
            pltpu.VMEM((emb.shape[0], d), jnp.float32),
            pltpu.VMEM((rows_per_w,), jnp.int32),
            [pltpu.VMEM((CHUNK, d), jnp.float32) for _ in range(NBUF)],
            [pltpu.SemaphoreType.DMA for _ in range(NBUF)],
            [pltpu.SemaphoreType.DMA for _ in range(NBUF)],
        ],
    )(x2, idx, emb)
    return out.reshape(b, l, d)


# speedup vs baseline: 2.0327x; 1.0020x over previous
import jax
import jax.numpy as jnp
from jax import lax
from jax.experimental import pallas as pl
from jax.experimental.pallas import tpu as pltpu
from jax.experimental.pallas import tpu_sc as plsc

NC = 2
NS = 16
NW = NC * NS
LANES = 16
CHUNK = 128
NBUF = 4


def _sc_body(rows_per_w, n_chunks, d, x_hbm, idx_hbm, emb_hbm, out_hbm,
             emb_v, idx_all, bufs, in_sems, out_sems):
    wid = lax.axis_index("s") * NC + lax.axis_index("c")
    base = wid * rows_per_w
    lookahead = NBUF - 1

    pltpu.sync_copy(emb_hbm, emb_v)
    pltpu.sync_copy(idx_hbm.at[pl.ds(base, rows_per_w)], idx_all)

    def start_in(k, b):
        row0 = base + k * CHUNK
        pltpu.async_copy(x_hbm.at[pl.ds(row0, CHUNK)], bufs[b], in_sems[b])

    def wait_in(b):
        pltpu.make_async_copy(x_hbm.at[pl.ds(base, CHUNK)], bufs[b],
                              in_sems[b]).wait()

    def start_out(k, b):
        row0 = base + k * CHUNK
        pltpu.async_copy(bufs[b], out_hbm.at[pl.ds(row0, CHUNK)],
                         out_sems[b])

    def wait_out(b):
        pltpu.make_async_copy(bufs[b], out_hbm.at[pl.ds(base, CHUNK)],
                              out_sems[b]).wait()

    def compute(k, b):
        buf = bufs[b]
        idx0 = k * CHUNK

        @pl.loop(0, CHUNK // LANES)
        def _grp(g):
            idxv = jnp.maximum(idx_all[pl.ds(idx0 + g * LANES, LANES)], 0)
            for i in range(LANES):
                s = idxv[i]
                r = g * LANES + i
                sls = [pl.ds(j * LANES, LANES) for j in range(d // LANES)]
                es = [emb_v[s, sl] for sl in sls]
                for j, sl in enumerate(sls):
                    plsc.addupdate(buf.at[r, sl], es[j])

    def iter_body(k, b, static_tail):
        wait_in(b)
        compute(k, b)
        start_out(k, b)
        if static_tail:
            return
        nxt = k + lookahead
        bn = (b + lookahead) % NBUF

        @pl.when(nxt < n_chunks)
        def _():
            @pl.when(k >= 1)
            def _():
                wait_out(bn)

            start_in(nxt, bn)

    n_main = (n_chunks // NBUF) * NBUF

    for c in range(min(lookahead, n_chunks)):
        start_in(c, c % NBUF)

    @pl.loop(0, n_main, step=NBUF)
    def _kk(kk):
        for b in range(NBUF):
            iter_body(kk + b, b, False)

    for k in range(n_main, n_chunks):
        iter_body(k, k % NBUF, True)

    for i in range(min(NBUF, n_chunks)):
        wait_out((n_chunks - 1 - i) % NBUF)


def kernel(x, month_idx, emb):
    b, l, d = x.shape
    n_rows = b * l
    rows_per_w = n_rows // NW
    n_chunks = rows_per_w // CHUNK
    assert rows_per_w * NW == n_rows and n_chunks * CHUNK == rows_per_w

    x2 = x.reshape(n_rows, d)
    idx = month_idx.reshape(n_rows).astype(jnp.int32)

    mesh = plsc.VectorSubcoreMesh(core_axis_name="c", subcore_axis_name="s")
    body = lambda *refs: _sc_body(rows_per_w, n_chunks, d, *refs)
    out = pl.kernel(
        body,
        out_type=jax.ShapeDtypeStruct((n_rows, d), jnp.float32),
        mesh=mesh,
        scratch_types=[
            pltpu.VMEM((emb.shape[0], d), jnp.float32),
            pltpu.VMEM((rows_per_w,), jnp.int32),
            [pltpu.VMEM((CHUNK, d), jnp.float32) for _ in range(NBUF)],
            [pltpu.SemaphoreType.DMA for _ in range(NBUF)],
            [pltpu.SemaphoreType.DMA for _ in range(NBUF)],
        ],
    )(x2, idx, emb)
    return out.reshape(b, l, d)


# speedup vs baseline: 2.0606x; 1.0137x over previous
import jax
import jax.numpy as jnp
from jax import lax
from jax.experimental import pallas as pl
from jax.experimental.pallas import tpu as pltpu
from jax.experimental.pallas import tpu_sc as plsc

NC = 2
NS = 16
NW = NC * NS
LANES = 16
CHUNK = 80
NBUF = 4


def _sc_body(rows_per_w, n_chunks, d, x_hbm, idx_hbm, emb_hbm, out_hbm,
             emb_v, idx_all, bufs, in_sems, out_sems):
    wid = lax.axis_index("s") * NC + lax.axis_index("c")
    base = wid * rows_per_w
    lookahead = NBUF - 1

    pltpu.sync_copy(emb_hbm, emb_v)
    pltpu.sync_copy(idx_hbm.at[pl.ds(base, rows_per_w)], idx_all)

    def start_in(k, b):
        row0 = base + k * CHUNK
        pltpu.async_copy(x_hbm.at[pl.ds(row0, CHUNK)], bufs[b], in_sems[b])

    def wait_in(b):
        pltpu.make_async_copy(x_hbm.at[pl.ds(base, CHUNK)], bufs[b],
                              in_sems[b]).wait()

    def start_out(k, b):
        row0 = base + k * CHUNK
        pltpu.async_copy(bufs[b], out_hbm.at[pl.ds(row0, CHUNK)],
                         out_sems[b])

    def wait_out(b):
        pltpu.make_async_copy(bufs[b], out_hbm.at[pl.ds(base, CHUNK)],
                              out_sems[b]).wait()

    def compute(k, b):
        buf = bufs[b]
        idx0 = k * CHUNK

        @pl.loop(0, CHUNK // LANES)
        def _grp(g):
            idxv = jnp.maximum(idx_all[pl.ds(idx0 + g * LANES, LANES)], 0)
            for i in range(LANES):
                s = idxv[i]
                r = g * LANES + i
                sls = [pl.ds(j * LANES, LANES) for j in range(d // LANES)]
                es = [emb_v[s, sl] for sl in sls]
                for j, sl in enumerate(sls):
                    plsc.addupdate(buf.at[r, sl], es[j])

    def iter_body(k, b, static_tail):
        wait_in(b)
        compute(k, b)
        start_out(k, b)
        if static_tail:
            return
        nxt = k + lookahead
        bn = (b + lookahead) % NBUF

        @pl.when(nxt < n_chunks)
        def _():
            @pl.when(k >= 1)
            def _():
                wait_out(bn)

            start_in(nxt, bn)

    n_main = (n_chunks // NBUF) * NBUF

    for c in range(min(lookahead, n_chunks)):
        start_in(c, c % NBUF)

    @pl.loop(0, n_main, step=NBUF)
    def _kk(kk):
        for b in range(NBUF):
            iter_body(kk + b, b, False)

    for k in range(n_main, n_chunks):
        iter_body(k, k % NBUF, True)

    for i in range(min(NBUF, n_chunks)):
        wait_out((n_chunks - 1 - i) % NBUF)


def kernel(x, month_idx, emb):
    b, l, d = x.shape
    n_rows = b * l
    rows_per_w = n_rows // NW
    n_chunks = rows_per_w // CHUNK
    assert rows_per_w * NW == n_rows and n_chunks * CHUNK == rows_per_w

    x2 = x.reshape(n_rows, d)
    idx = month_idx.reshape(n_rows).astype(jnp.int32)

    mesh = plsc.VectorSubcoreMesh(core_axis_name="c", subcore_axis_name="s")
    body = lambda *refs: _sc_body(rows_per_w, n_chunks, d, *refs)
    out = pl.kernel(
        body,
        out_type=jax.ShapeDtypeStruct((n_rows, d), jnp.float32),
        mesh=mesh,
        scratch_types=[
            pltpu.VMEM((emb.shape[0], d), jnp.float32),
            pltpu.VMEM((rows_per_w,), jnp.int32),
            [pltpu.VMEM((CHUNK, d), jnp.float32) for _ in range(NBUF)],
            [pltpu.SemaphoreType.DMA for _ in range(NBUF)],
            [pltpu.SemaphoreType.DMA for _ in range(NBUF)],
        ],
    )(x2, idx, emb)
    return out.reshape(b, l, d)
